# baseline (speedup 1.0000x reference)
import jax
import jax.numpy as jnp
from jax import lax
from jax.experimental import pallas as pl
from jax.experimental.pallas import tpu as pltpu

_B, _G, _E = 4, 64, 8
_DS, _DE, _H, _A = 1024, 64, 1024, 16
_DIN = _DS + _DE
_N = _B * _G
_T = 32
_NPAD = _N + _T


def _mlp_kernel(counts_ref, starts_ref, perm_ref, state_ref, emb_ref,
                w1_ref, b1_ref, w2_ref, b2_ref, w3_ref, b3_ref, out_ref):
    e = pl.program_id(0)
    bf = jnp.bfloat16
    f32 = jnp.float32
    w1 = w1_ref[0]
    sp = jnp.dot(state_ref[...].astype(bf), w1[_DE:, :].astype(bf),
                 preferred_element_type=f32)
    ep = jnp.dot(emb_ref[...].astype(bf), w1[:_DE, :].astype(bf),
                 preferred_element_type=f32)
    w2b = w2_ref[0].astype(bf)
    w3b = w3_ref[0].astype(bf)
    b1v = b1_ref[0]
    b2v = b2_ref[0]
    b3v = b3_ref[0]

    @pl.when(e == 0)
    def _():
        out_ref[...] = jnp.zeros_like(out_ref)

    count = counts_ref[e]
    start = starts_ref[e]
    nchunks = (count + _T - 1) // _T

    def body(j, carry):
        base = start + j * _T
        tid = perm_ref[pl.ds(base, _T), :]
        riota = lax.broadcasted_iota(jnp.int32, (_T, 1), 0)
        valid = (j * _T + riota) < count
        bidx = tid // _G
        gidx = tid - bidx * _G
        oh_b = (bidx == lax.broadcasted_iota(jnp.int32, (_T, _B), 1)).astype(f32)
        oh_g = (gidx == lax.broadcasted_iota(jnp.int32, (_T, _G), 1)).astype(f32)
        h1c = jnp.maximum(
            jnp.dot(oh_b, sp, preferred_element_type=f32)
            + jnp.dot(oh_g, ep, preferred_element_type=f32) + b1v, 0.0)
        h2c = jnp.maximum(
            jnp.dot(h1c.astype(bf), w2b, preferred_element_type=f32) + b2v,
            0.0)
        oc = (jnp.dot(h2c.astype(bf), w3b, preferred_element_type=f32)
              + b3v)
        oh_t = ((tid == lax.broadcasted_iota(jnp.int32, (_T, _N), 1))
                & valid).astype(f32)
        out_ref[...] += lax.dot_general(
            oh_t, oc, (((0,), (0,)), ((), ())), preferred_element_type=f32)
        return carry

    lax.fori_loop(0, nchunks, body, 0)


def _run_mlp(perm, counts, starts, state, agent_emb, W1, b1, W2, b2, W3, b3):
    return pl.pallas_call(
        _mlp_kernel,
        grid=(_E,),
        in_specs=[
            pl.BlockSpec(memory_space=pltpu.SMEM),
            pl.BlockSpec(memory_space=pltpu.SMEM),
            pl.BlockSpec((_NPAD, 1), lambda e: (0, 0)),
            pl.BlockSpec((_B, _DS), lambda e: (0, 0)),
            pl.BlockSpec((_G, _DE), lambda e: (0, 0)),
            pl.BlockSpec((1, _DIN, _H), lambda e: (e, 0, 0)),
            pl.BlockSpec((1, 1, _H), lambda e: (e, 0, 0)),
            pl.BlockSpec((1, _H, _H), lambda e: (e, 0, 0)),
            pl.BlockSpec((1, 1, _H), lambda e: (e, 0, 0)),
            pl.BlockSpec((1, _H, _A), lambda e: (e, 0, 0)),
            pl.BlockSpec((1, 1, _A), lambda e: (e, 0, 0)),
        ],
        out_specs=pl.BlockSpec((_N, _A), lambda e: (0, 0)),
        out_shape=jax.ShapeDtypeStruct((_N, _A), jnp.float32),
        compiler_params=pltpu.CompilerParams(
            dimension_semantics=("arbitrary",)),
    )(counts, starts, perm, state, agent_emb, W1, b1.reshape(_E, 1, _H), W2,
      b2.reshape(_E, 1, _H), W3, b3.reshape(_E, 1, _A))


def kernel(state, assigner_logits, agent_emb, W1, b1, W2, b2, W3, b3):
    u = jax.random.uniform(jax.random.key(1), (_B, _G, _E), jnp.float32,
                           1e-6, 1.0 - 1e-6)
    gumbel = -jnp.log(-jnp.log(u))
    scores = assigner_logits[None, :, :] + gumbel
    eidx = jnp.argmax(scores, axis=-1).reshape(_N).astype(jnp.int32)
    perm = jnp.argsort(eidx, stable=True).astype(jnp.int32)
    counts = jnp.sum(
        (eidx[:, None] == jnp.arange(_E)[None, :]).astype(jnp.int32), axis=0)
    starts = jnp.concatenate(
        [jnp.zeros((1,), jnp.int32), jnp.cumsum(counts)[:-1]])
    perm_pad = jnp.pad(perm, (0, _NPAD - _N)).reshape(_NPAD, 1)
    out = _run_mlp(perm_pad, counts, starts, state, agent_emb, W1, b1, W2, b2,
                   W3, b3)
    return out.reshape(_B, _G, _A)


# speedup vs baseline: 1.0611x; 1.0611x over previous
import jax
import jax.numpy as jnp
from jax import lax
from jax.experimental import pallas as pl
from jax.experimental.pallas import tpu as pltpu

_B, _G, _E = 4, 64, 8
_DS, _DE, _H, _A = 1024, 64, 1024, 16
_DIN = _DS + _DE
_N = _B * _G
_T = 32
_NCHUNK = _N // _T


def _mlp_kernel(counts_ref, perm_ref, state_ref, emb_ref,
                w1_ref, b1_ref, w2_ref, b2_ref, w3_ref, b3_ref, out_ref):
    e = pl.program_id(0)
    bf = jnp.bfloat16
    f32 = jnp.float32
    w1 = w1_ref[0]
    sp = jnp.dot(state_ref[...].astype(bf), w1[_DE:, :].astype(bf),
                 preferred_element_type=f32)
    ep = jnp.dot(emb_ref[...].astype(bf), w1[:_DE, :].astype(bf),
                 preferred_element_type=f32)
    w2b = w2_ref[0].astype(bf)
    w3b = w3_ref[0].astype(bf)
    b1v = b1_ref[0]
    b2v = b2_ref[0]
    b3v = b3_ref[0]

    @pl.when(e == 0)
    def _():
        out_ref[...] = jnp.zeros_like(out_ref)

    count = counts_ref[e]

    for j in range(_NCHUNK):
        @pl.when(j * _T < count)
        def _(j=j):
            tid = perm_ref[0, pl.ds(j * _T, _T), :]
            riota = lax.broadcasted_iota(jnp.int32, (_T, 1), 0)
            valid = (j * _T + riota) < count
            bidx = tid // _G
            gidx = tid - bidx * _G
            oh_b = (bidx == lax.broadcasted_iota(jnp.int32, (_T, _B), 1)
                    ).astype(f32)
            oh_g = (gidx == lax.broadcasted_iota(jnp.int32, (_T, _G), 1)
                    ).astype(f32)
            h1c = jnp.maximum(
                jnp.dot(oh_b, sp, preferred_element_type=f32)
                + jnp.dot(oh_g, ep, preferred_element_type=f32) + b1v, 0.0)
            h2c = jnp.maximum(
                jnp.dot(h1c.astype(bf), w2b, preferred_element_type=f32)
                + b2v, 0.0)
            oc = (jnp.dot(h2c.astype(bf), w3b, preferred_element_type=f32)
                  + b3v)
            oh_t = ((tid == lax.broadcasted_iota(jnp.int32, (_T, _N), 1))
                    & valid).astype(f32)
            out_ref[...] += lax.dot_general(
                oh_t, oc, (((0,), (0,)), ((), ())),
                preferred_element_type=f32)


def _run_mlp(perm, counts, state, agent_emb, W1, b1, W2, b2, W3, b3):
    return pl.pallas_call(
        _mlp_kernel,
        grid=(_E,),
        in_specs=[
            pl.BlockSpec(memory_space=pltpu.SMEM),
            pl.BlockSpec((1, _N, 1), lambda e: (e, 0, 0)),
            pl.BlockSpec((_B, _DS), lambda e: (0, 0)),
            pl.BlockSpec((_G, _DE), lambda e: (0, 0)),
            pl.BlockSpec((1, _DIN, _H), lambda e: (e, 0, 0)),
            pl.BlockSpec((1, 1, _H), lambda e: (e, 0, 0)),
            pl.BlockSpec((1, _H, _H), lambda e: (e, 0, 0)),
            pl.BlockSpec((1, 1, _H), lambda e: (e, 0, 0)),
            pl.BlockSpec((1, _H, _A), lambda e: (e, 0, 0)),
            pl.BlockSpec((1, 1, _A), lambda e: (e, 0, 0)),
        ],
        out_specs=pl.BlockSpec((_N, _A), lambda e: (0, 0)),
        out_shape=jax.ShapeDtypeStruct((_N, _A), jnp.float32),
        compiler_params=pltpu.CompilerParams(
            dimension_semantics=("arbitrary",)),
    )(counts, perm, state, agent_emb, W1, b1.reshape(_E, 1, _H), W2,
      b2.reshape(_E, 1, _H), W3, b3.reshape(_E, 1, _A))


def _route(assigner_logits):
    u = jax.random.uniform(jax.random.key(1), (_B, _G, _E), jnp.float32,
                           1e-6, 1.0 - 1e-6)
    gumbel = -jnp.log(-jnp.log(u))
    scores = assigner_logits[None, :, :] + gumbel
    eidx = jnp.argmax(scores, axis=-1).reshape(_N).astype(jnp.int32)
    oh = (eidx[:, None] == jnp.arange(_E)[None, :]).astype(jnp.float32)
    counts = jnp.sum(oh, axis=0).astype(jnp.int32)
    tri = jnp.tril(jnp.ones((_N, _N), jnp.float32))
    csum = jnp.dot(tri, oh, preferred_element_type=jnp.float32)
    rank = jnp.sum(csum * oh, axis=1) - 1.0
    slot_oh = (rank[None, :] == jnp.arange(_N, dtype=jnp.float32)[:, None]
               ).astype(jnp.float32)
    tok_oh = jnp.arange(_N, dtype=jnp.float32)[:, None] * oh
    perm = jnp.dot(slot_oh, tok_oh,
                   preferred_element_type=jnp.float32)
    perm = perm.astype(jnp.int32).T.reshape(_E, _N, 1)
    return perm, counts


def kernel(state, assigner_logits, agent_emb, W1, b1, W2, b2, W3, b3):
    perm, counts = _route(assigner_logits)
    out = _run_mlp(perm, counts, state, agent_emb, W1, b1, W2, b2, W3, b3)
    return out.reshape(_B, _G, _A)
